# baseline (speedup 1.0000x reference)
import jax
import jax.numpy as jnp
from jax import lax
from jax.experimental import pallas as pl
from jax.experimental.pallas import tpu as pltpu
from jax.experimental.pallas import tpu_sc as plsc

BATCH = 4096
SEQ = 200
SEQ_PAD = 208
HALF = SEQ_PAD // 2
EMBED = 64
NUM_WORKERS = 32
ROWS_PER_W = BATCH // NUM_WORKERS
LANES = 16
D_CH = EMBED // LANES


def _body(idx_hbm, w_hbm, out_hbm, idx_v, rows_v, out_v, sem):
    wid = lax.axis_index("s") * 2 + lax.axis_index("c")
    base = wid * ROWS_PER_W
    pltpu.sync_copy(idx_hbm.at[pl.ds(base, ROWS_PER_W)], idx_v)

    def row_body(b, carry):
        cp0 = pltpu.async_copy(w_hbm.at[idx_v.at[b, pl.ds(0, HALF)]],
                               rows_v.at[pl.ds(0, HALF)], sem)
        cp1 = pltpu.async_copy(w_hbm.at[idx_v.at[b, pl.ds(HALF, HALF)]],
                               rows_v.at[pl.ds(HALF, HALF)], sem)

        cnt = jnp.zeros((LANES,), jnp.int32)
        for c in range(SEQ_PAD // LANES):
            iv = idx_v[b, pl.ds(c * LANES, LANES)]
            cnt = cnt + plsc.all_reduce_population_count(iv != 0)
        inv = 1.0 / jnp.maximum(cnt.astype(jnp.float32), 1.0)

        cp0.wait()
        cp1.wait()

        def inner(i, accs):
            return tuple(accs[d] + rows_v[i, pl.ds(d * LANES, LANES)]
                         for d in range(D_CH))

        zeros = tuple(jnp.zeros((LANES,), jnp.float32) for _ in range(D_CH))
        accs = lax.fori_loop(0, SEQ_PAD, inner, zeros, unroll=8)
        for d in range(D_CH):
            out_v[b, pl.ds(d * LANES, LANES)] = accs[d] * inv
        return carry

    lax.fori_loop(0, ROWS_PER_W, row_body, 0)
    pltpu.sync_copy(out_v, out_hbm.at[pl.ds(base, ROWS_PER_W)])


def kernel(sentence, weight):
    s32 = sentence.astype(jnp.int32)
    idx = jnp.concatenate(
        [s32, jnp.zeros((s32.shape[0], SEQ_PAD - SEQ), jnp.int32)], axis=1)
    f = pl.kernel(
        _body,
        out_type=jax.ShapeDtypeStruct((BATCH, EMBED), jnp.float32),
        mesh=plsc.VectorSubcoreMesh(core_axis_name="c", subcore_axis_name="s"),
        scratch_types=[
            pltpu.VMEM((ROWS_PER_W, SEQ_PAD), jnp.int32),
            pltpu.VMEM((SEQ_PAD, EMBED), jnp.float32),
            pltpu.VMEM((ROWS_PER_W, EMBED), jnp.float32),
            pltpu.SemaphoreType.DMA,
        ],
        compiler_params=pltpu.CompilerParams(use_tc_tiling_on_sc=False,
                                             needs_layout_passes=False),
    )
    return f(idx, weight)
